# baseline (speedup 1.0000x reference)
import functools

import jax
import jax.numpy as jnp
from jax import lax
from jax.experimental import pallas as pl
from jax.experimental.pallas import tpu as pltpu
from jax.experimental.pallas import tpu_sc as plsc

B = 16
S = 512
D = 768
NCAND = 32
TPAD = B * S
HALF = TPAD // 2
BLK = 512
NBLK_H = HALF // BLK
NW = 32
CWH_MAX = HALF // NW


@functools.lru_cache(maxsize=None)
def _make_sc_gather(half):
    mesh = plsc.VectorSubcoreMesh(core_axis_name="c", subcore_axis_name="s")

    out_type = (
        jax.ShapeDtypeStruct((HALF, D), jnp.float32),
        jax.ShapeDtypeStruct((B * NCAND, D), jnp.float32),
        jax.ShapeDtypeStruct((B * NCAND,), jnp.float32),
        jax.ShapeDtypeStruct((B, 2), jnp.int32),
        jax.ShapeDtypeStruct((B, 1), jnp.float32),
    )

    @functools.partial(
        pl.kernel,
        mesh=mesh,
        compiler_params=pltpu.CompilerParams(needs_layout_passes=False),
        out_type=out_type,
        scratch_types=[
            pltpu.VMEM((B * S,), jnp.int32),
            pltpu.VMEM((32,), jnp.int32),
            pltpu.VMEM((48,), jnp.int32),
            pltpu.VMEM((B, 2), jnp.int32),
            pltpu.VMEM((B, 1), jnp.float32),
            pltpu.VMEM((64,), jnp.int32),
            pltpu.VMEM((64, D), jnp.float32),
            pltpu.VMEM((64,), jnp.int32),
            pltpu.VMEM((64, D), jnp.float32),
            pltpu.VMEM((16,), jnp.int32),
            pltpu.VMEM((16,), jnp.int32),
            pltpu.VMEM((16,), jnp.int32),
            pltpu.VMEM((16, D), jnp.float32),
            pltpu.VMEM((16, 128), jnp.float32),
            pltpu.VMEM((16,), jnp.float32),
            pltpu.SemaphoreType.DMA,
            pltpu.SemaphoreType.DMA,
            pltpu.SemaphoreType.DMA,
            pltpu.SemaphoreType.DMA,
            pltpu.SemaphoreType.DMA,
            pltpu.SemaphoreType.DMA,
        ],
    )
    def sc_gather(ctx_hbm, spans_hbm, sids_hbm, emb_hbm, sw_hbm, sb_hbm,
                  tok_out, wg_out, bg_out, lohi_out, iw_out,
                  ctx_v, spans_v, aux_v, lohi_s, iw_s,
                  ids_a, rows_a, ids_b, rows_b, ids_t,
                  sidx_v, sidx_hi_v, srows_v, sbrows_v, sb_v,
                  sem_sw, sem_sb, semg_a, semg_b, semo_a, semo_b):
        wid = lax.axis_index("s") * 2 + lax.axis_index("c")
        sbase = pl.multiple_of(wid * 16, 16)
        lane16 = lax.iota(jnp.int32, 16)

        if half == 0:
            pltpu.sync_copy(sids_hbm.at[pl.ds(sbase, 16)], sidx_v)
            sids = sidx_v[...]
            sidx_hi_v[...] = jnp.right_shift(sids, 7)
            pltpu.async_copy(sw_hbm.at[sidx_v], srows_v, sem_sw)
            pltpu.async_copy(sb_hbm.at[sidx_hi_v], sbrows_v, sem_sb)

        pltpu.sync_copy(spans_hbm, spans_v)
        st_v = plsc.load_gather(spans_v, [lane16 * 2])
        en_v = plsc.load_gather(spans_v, [lane16 * 2 + 1])
        en_v = jnp.maximum(en_v, st_v + 1)
        w_v = en_v - st_v
        cumhi_v = plsc.cumsum(w_v)
        cumlo_v = cumhi_v - w_v
        t_total = jnp.max(cumhi_v)
        tp = ((t_total + 1023) // 1024) * 1024
        cw = tp // (2 * NW)
        base = pl.multiple_of(wid * cw, 16)
        toff = half * (tp // 2) + base
        aux_v[pl.ds(0, 16)] = st_v
        aux_v[pl.ds(16, 16)] = cumlo_v
        aux_v[pl.ds(32, 16)] = cumhi_v

        if half == 0:
            @pl.when(wid == 0)
            def _():
                zeros16 = jnp.zeros((16,), jnp.int32)
                plsc.store_scatter(lohi_s, [lane16, zeros16], cumlo_v)
                plsc.store_scatter(lohi_s, [lane16, zeros16 + 1], cumhi_v)
                plsc.store_scatter(iw_s, [lane16, zeros16],
                                   1.0 / w_v.astype(jnp.float32))
                pltpu.sync_copy(lohi_s, lohi_out)
                pltpu.sync_copy(iw_s, iw_out)

        pltpu.sync_copy(ctx_hbm, ctx_v)

        nch64 = cw // 64
        rem16 = (cw - nch64 * 64) // 16
        npair = (nch64 + 1) // 2

        cum_hi = [plsc.load_gather(aux_v, [jnp.full((16,), 32 + j, jnp.int32)])
                  for j in range(B)]
        t_end = cum_hi[B - 1]

        def resolve_ids(t0):
            t_vec = t0 + lane16
            b_vec = jnp.zeros((16,), jnp.int32)
            for j in range(B):
                b_vec = b_vec + (t_vec >= cum_hi[j]).astype(jnp.int32)
            b_vec = jnp.minimum(b_vec, B - 1)
            st = plsc.load_gather(aux_v, [b_vec])
            cl = plsc.load_gather(aux_v, [b_vec + 16])
            pos = st + (t_vec - cl) + b_vec * S
            pos = jnp.where(t_vec < t_end, pos, 0)
            return plsc.load_gather(ctx_v, [pos])

        def build_ids(dst, k):
            for i in range(4):
                dst[pl.ds(i * 16, 16)] = resolve_ids(toff + k * 64 + i * 16)

        def out64(k):
            return tok_out.at[pl.ds(pl.multiple_of(base + k * 64, 16), 64)]

        def pair_body(p, carry):
            k0 = p * 2
            k1 = k0 + 1

            @pl.when(p > 0)
            def _():
                pltpu.make_async_copy(rows_a, out64(k0), semo_a).wait()
                pltpu.make_async_copy(rows_b, out64(k0), semo_b).wait()

            build_ids(ids_a, k0)
            pltpu.async_copy(emb_hbm.at[ids_a], rows_a, semg_a)

            @pl.when(k1 < nch64)
            def _():
                build_ids(ids_b, k1)
                pltpu.async_copy(emb_hbm.at[ids_b], rows_b, semg_b)

            pltpu.make_async_copy(emb_hbm.at[ids_a], rows_a, semg_a).wait()
            pltpu.async_copy(rows_a, out64(k0), semo_a)

            @pl.when(k1 < nch64)
            def _():
                pltpu.make_async_copy(emb_hbm.at[ids_b], rows_b, semg_b).wait()
                pltpu.async_copy(rows_b, out64(k1), semo_b)

            return carry

        lax.fori_loop(0, npair, pair_body, 0)

        @pl.when(nch64 > 0)
        def _():
            pltpu.make_async_copy(rows_a, out64(0), semo_a).wait()

        @pl.when((nch64 > 0) & (nch64 == (nch64 // 2) * 2))
        def _():
            pltpu.make_async_copy(rows_b, out64(0), semo_b).wait()

        j0 = nch64 * 64

        def body16(j, carry):
            ids_t[...] = resolve_ids(toff + j0 + j * 16)
            rows_t = rows_a.at[pl.ds(0, 16)]
            pltpu.async_copy(emb_hbm.at[ids_t], rows_t, semg_a).wait()
            pltpu.sync_copy(
                rows_t,
                tok_out.at[pl.ds(pl.multiple_of(base + j0 + j * 16, 16), 16)])
            return carry

        lax.fori_loop(0, rem16, body16, 0)

        if half == 0:
            pltpu.make_async_copy(sw_hbm.at[sidx_v], srows_v, sem_sw).wait()
            pltpu.sync_copy(srows_v, wg_out.at[pl.ds(sbase, 16)])
            pltpu.make_async_copy(sb_hbm.at[sidx_hi_v], sbrows_v, sem_sb).wait()
            sb_v[...] = plsc.load_gather(
                sbrows_v, [lane16, jnp.bitwise_and(sids, 127)])
            pltpu.sync_copy(sb_v, bg_out.at[pl.ds(sbase, 16)])

    return sc_gather


def _sc_gather_h0(*args):
    return _make_sc_gather(0)(*args)


def _sc_gather_h1(*args):
    return _make_sc_gather(1)(*args)


def _nbh(sp_ref):
    t_total = jnp.int32(0)
    for b in range(B):
        s_b = sp_ref[2 * b]
        e_b = jnp.maximum(sp_ref[2 * b + 1], s_b + 1)
        t_total = t_total + (e_b - s_b)
    return (t_total + 1023) // 1024


def _mask_weights(i_glob, lohi_ref, iw_ref):
    gcol = i_glob * BLK + lax.broadcasted_iota(jnp.int32, (B, BLK), 1)
    lo = lohi_ref[:, 0:1]
    hi = lohi_ref[:, 1:2]
    m = ((gcol >= lo) & (gcol < hi)).astype(jnp.float32)
    return m * iw_ref[...]


def _tc_body_h1(sp_ref, tok_ref, w_ref, b_ref, lohi_ref, iw_ref, acc_out):
    i = pl.program_id(0)
    nbh = _nbh(sp_ref)

    @pl.when(i == 0)
    def _():
        acc_out[...] = jnp.zeros_like(acc_out)

    @pl.when(i < nbh)
    def _():
        h = jnp.tanh(
            jnp.dot(tok_ref[...], w_ref[...], preferred_element_type=jnp.float32)
            + b_ref[...])
        m = _mask_weights(i, lohi_ref, iw_ref)
        acc_out[...] += jnp.dot(m, h, preferred_element_type=jnp.float32)


def _tc_body_h2(sp_ref, tok_ref, w_ref, b_ref, lohi_ref, iw_ref, acc1_ref,
                wg_ref, bgr_ref, tgt_ref, loss_ref, corr_ref, acc_s):
    i = pl.program_id(0)
    nbh = _nbh(sp_ref)

    @pl.when(i == 0)
    def _():
        acc_s[...] = jnp.zeros_like(acc_s)

    @pl.when(i < nbh)
    def _():
        h = jnp.tanh(
            jnp.dot(tok_ref[...], w_ref[...], preferred_element_type=jnp.float32)
            + b_ref[...])
        m = _mask_weights(nbh + i, lohi_ref, iw_ref)
        acc_s[...] += jnp.dot(m, h, preferred_element_type=jnp.float32)

    @pl.when(i == NBLK_H - 1)
    def _():
        reps = acc1_ref[...] + acc_s[...]
        rows = []
        for b in range(B):
            wgb = wg_ref[pl.ds(b * NCAND, NCAND), :]
            rb = reps[b:b + 1, :]
            rows.append(lax.dot_general(
                rb, wgb, (((1,), (1,)), ((), ())),
                preferred_element_type=jnp.float32))
        logits = jnp.concatenate(rows, axis=0) + bgr_ref[...]

        mx = jnp.max(logits, axis=1, keepdims=True)
        ex = jnp.exp(logits - mx)
        z = jnp.sum(ex, axis=1, keepdims=True)
        logz = jnp.log(z) + mx
        ci = lax.broadcasted_iota(jnp.int32, (B, NCAND), 1)
        tgt = tgt_ref[...]
        tl = jnp.sum(jnp.where(ci == tgt, logits, 0.0), axis=1, keepdims=True)
        loss_ref[...] = jnp.sum((logz - tl) * (1.0 / B), axis=0, keepdims=True)
        amax = jnp.min(jnp.where(logits == mx, ci, NCAND), axis=1, keepdims=True)
        corr_ref[...] = (amax == tgt).astype(jnp.int32)


def _tok_map(i, sp):
    return (jnp.minimum(i, _nbh(sp) - 1), 0)


def _zero_map(i, sp):
    return (0, 0)


def _tc_half1(spans_flat, tok1, w_enc, b_enc2, lohi, iw):
    grid_spec = pltpu.PrefetchScalarGridSpec(
        num_scalar_prefetch=1,
        grid=(NBLK_H,),
        in_specs=[
            pl.BlockSpec((BLK, D), _tok_map),
            pl.BlockSpec((D, D), _zero_map),
            pl.BlockSpec((1, D), _zero_map),
            pl.BlockSpec((B, 2), _zero_map),
            pl.BlockSpec((B, 1), _zero_map),
        ],
        out_specs=pl.BlockSpec((B, D), _zero_map),
        scratch_shapes=[],
    )
    return pl.pallas_call(
        _tc_body_h1,
        grid_spec=grid_spec,
        out_shape=jax.ShapeDtypeStruct((B, D), jnp.float32),
    )(spans_flat, tok1, w_enc, b_enc2, lohi, iw)


def _tc_half2(spans_flat, tok2, w_enc, b_enc2, lohi, iw, acc1, wg, bgr, tgt2):
    grid_spec = pltpu.PrefetchScalarGridSpec(
        num_scalar_prefetch=1,
        grid=(NBLK_H,),
        in_specs=[
            pl.BlockSpec((BLK, D), _tok_map),
            pl.BlockSpec((D, D), _zero_map),
            pl.BlockSpec((1, D), _zero_map),
            pl.BlockSpec((B, 2), _zero_map),
            pl.BlockSpec((B, 1), _zero_map),
            pl.BlockSpec((B, D), _zero_map),
            pl.BlockSpec((B * NCAND, D), _zero_map),
            pl.BlockSpec((B, NCAND), _zero_map),
            pl.BlockSpec((B, 1), _zero_map),
        ],
        out_specs=[
            pl.BlockSpec((1, 1), _zero_map),
            pl.BlockSpec((B, 1), _zero_map),
        ],
        scratch_shapes=[pltpu.VMEM((B, D), jnp.float32)],
    )
    return pl.pallas_call(
        _tc_body_h2,
        grid_spec=grid_spec,
        out_shape=[
            jax.ShapeDtypeStruct((1, 1), jnp.float32),
            jax.ShapeDtypeStruct((B, 1), jnp.int32),
        ],
    )(spans_flat, tok2, w_enc, b_enc2, lohi, iw, acc1, wg, bgr, tgt2)


def kernel(context_ids, context_spans, sense_ids, target_ids, emb_table,
           W_enc, b_enc, sense_W, sense_b):
    context_ids = context_ids.astype(jnp.int32)
    context_spans = context_spans.astype(jnp.int32)
    sense_ids = sense_ids.astype(jnp.int32)
    target_ids = target_ids.astype(jnp.int32)

    spans_flat = context_spans.reshape(-1)
    ctx_flat = context_ids.reshape(-1)
    sids_flat = sense_ids.reshape(-1)

    n_senses = sense_b.shape[0]
    pad_b = (-n_senses) % 128
    sb_rows = jnp.pad(sense_b, (0, pad_b)).reshape(-1, 128)

    tok1, wg, bg, lohi, iw = _sc_gather_h0(ctx_flat, spans_flat, sids_flat,
                                           emb_table, sense_W, sb_rows)
    tok2 = _sc_gather_h1(ctx_flat, spans_flat, sids_flat,
                         emb_table, sense_W, sb_rows)[0]

    b_enc2 = b_enc.reshape(1, D)
    acc1 = _tc_half1(spans_flat, tok1, W_enc, b_enc2, lohi, iw)
    loss2, corr2 = _tc_half2(spans_flat, tok2, W_enc, b_enc2, lohi, iw, acc1,
                             wg, bg.reshape(B, NCAND), target_ids.reshape(B, 1))
    return loss2[0, 0], corr2[:, 0].astype(jnp.bool_)


# speedup vs baseline: 1.0809x; 1.0809x over previous
import functools

import jax
import jax.numpy as jnp
from jax import lax
from jax.experimental import pallas as pl
from jax.experimental.pallas import tpu as pltpu
from jax.experimental.pallas import tpu_sc as plsc

B = 16
S = 512
D = 768
NCAND = 32
TPAD = B * S
BLK = 512
NBLK = TPAD // BLK
NW = 32
CW_MAX = TPAD // NW

@functools.lru_cache(maxsize=None)
def _make_sc_gather():
    mesh = plsc.VectorSubcoreMesh(core_axis_name="c", subcore_axis_name="s")

    @functools.partial(
        pl.kernel,
        mesh=mesh,
        compiler_params=pltpu.CompilerParams(needs_layout_passes=False),
        out_type=(
            jax.ShapeDtypeStruct((TPAD, D), jnp.float32),
            jax.ShapeDtypeStruct((B * NCAND, D), jnp.float32),
            jax.ShapeDtypeStruct((B * NCAND,), jnp.float32),
            jax.ShapeDtypeStruct((B, 2), jnp.int32),
            jax.ShapeDtypeStruct((B, 1), jnp.float32),
        ),
        scratch_types=[
            pltpu.VMEM((B * S,), jnp.int32),
            pltpu.VMEM((32,), jnp.int32),
            pltpu.VMEM((48,), jnp.int32),
            pltpu.VMEM((B, 2), jnp.int32),
            pltpu.VMEM((B, 1), jnp.float32),
            pltpu.VMEM((64,), jnp.int32),
            pltpu.VMEM((64, D), jnp.float32),
            pltpu.VMEM((64,), jnp.int32),
            pltpu.VMEM((64, D), jnp.float32),
            pltpu.VMEM((16,), jnp.int32),
            pltpu.VMEM((16,), jnp.int32),
            pltpu.VMEM((16,), jnp.int32),
            pltpu.VMEM((16, D), jnp.float32),
            pltpu.VMEM((16, 128), jnp.float32),
            pltpu.VMEM((16,), jnp.float32),
            pltpu.SemaphoreType.DMA,
            pltpu.SemaphoreType.DMA,
            pltpu.SemaphoreType.DMA,
            pltpu.SemaphoreType.DMA,
            pltpu.SemaphoreType.DMA,
            pltpu.SemaphoreType.DMA,
        ],
    )
    def sc_gather(ctx_hbm, spans_hbm, sids_hbm, emb_hbm, sw_hbm, sb_hbm,
                  tok_out, wg_out, bg_out, lohi_out, iw_out,
                  ctx_v, spans_v, aux_v, lohi_s, iw_s,
                  ids_a, rows_a, ids_b, rows_b, ids_t,
                  sidx_v, sidx_hi_v, srows_v, sbrows_v, sb_v,
                  sem_sw, sem_sb, semg_a, semg_b, semo_a, semo_b):
        wid = lax.axis_index("s") * 2 + lax.axis_index("c")
        sbase = pl.multiple_of(wid * 16, 16)
        lane16 = lax.iota(jnp.int32, 16)

        pltpu.sync_copy(sids_hbm.at[pl.ds(sbase, 16)], sidx_v)
        sids = sidx_v[...]
        sidx_hi_v[...] = jnp.right_shift(sids, 7)
        pltpu.async_copy(sw_hbm.at[sidx_v], srows_v, sem_sw)
        pltpu.async_copy(sb_hbm.at[sidx_hi_v], sbrows_v, sem_sb)

        pltpu.sync_copy(spans_hbm, spans_v)
        st_v = plsc.load_gather(spans_v, [lane16 * 2])
        en_v = plsc.load_gather(spans_v, [lane16 * 2 + 1])
        en_v = jnp.maximum(en_v, st_v + 1)
        w_v = en_v - st_v
        cumhi_v = plsc.cumsum(w_v)
        cumlo_v = cumhi_v - w_v
        t_total = jnp.max(cumhi_v)
        tp = ((t_total + 511) // 512) * 512
        cw = tp // NW
        base = pl.multiple_of(wid * cw, 16)
        aux_v[pl.ds(0, 16)] = st_v
        aux_v[pl.ds(16, 16)] = cumlo_v
        aux_v[pl.ds(32, 16)] = cumhi_v

        @pl.when(wid == 0)
        def _():
            zeros16 = jnp.zeros((16,), jnp.int32)
            plsc.store_scatter(lohi_s, [lane16, zeros16], cumlo_v)
            plsc.store_scatter(lohi_s, [lane16, zeros16 + 1], cumhi_v)
            plsc.store_scatter(iw_s, [lane16, zeros16],
                               1.0 / w_v.astype(jnp.float32))
            pltpu.sync_copy(lohi_s, lohi_out)
            pltpu.sync_copy(iw_s, iw_out)

        pltpu.sync_copy(ctx_hbm, ctx_v)

        nch64 = cw // 64
        rem16 = (cw - nch64 * 64) // 16
        npair = (nch64 + 1) // 2

        cum_hi = [plsc.load_gather(aux_v, [jnp.full((16,), 32 + j, jnp.int32)])
                  for j in range(B)]
        t_end = cum_hi[B - 1]

        def build_ids(dst, k):
            for i in range(4):
                t_vec = (base + k * 64 + i * 16) + lane16
                b_vec = jnp.zeros((16,), jnp.int32)
                for j in range(B):
                    b_vec = b_vec + (t_vec >= cum_hi[j]).astype(jnp.int32)
                b_vec = jnp.minimum(b_vec, B - 1)
                st = plsc.load_gather(aux_v, [b_vec])
                cl = plsc.load_gather(aux_v, [b_vec + 16])
                pos = st + (t_vec - cl) + b_vec * S
                pos = jnp.where(t_vec < t_end, pos, 0)
                dst[pl.ds(i * 16, 16)] = plsc.load_gather(ctx_v, [pos])

        def out64(k):
            return tok_out.at[pl.ds(pl.multiple_of(base + k * 64, 16), 64)]

        def pair_body(p, carry):
            k0 = p * 2
            k1 = k0 + 1

            @pl.when(p > 0)
            def _():
                pltpu.make_async_copy(rows_a, out64(k0), semo_a).wait()
                pltpu.make_async_copy(rows_b, out64(k0), semo_b).wait()

            build_ids(ids_a, k0)
            pltpu.async_copy(emb_hbm.at[ids_a], rows_a, semg_a)

            @pl.when(k1 < nch64)
            def _():
                build_ids(ids_b, k1)
                pltpu.async_copy(emb_hbm.at[ids_b], rows_b, semg_b)

            pltpu.make_async_copy(emb_hbm.at[ids_a], rows_a, semg_a).wait()
            pltpu.async_copy(rows_a, out64(k0), semo_a)

            @pl.when(k1 < nch64)
            def _():
                pltpu.make_async_copy(emb_hbm.at[ids_b], rows_b, semg_b).wait()
                pltpu.async_copy(rows_b, out64(k1), semo_b)

            return carry

        lax.fori_loop(0, npair, pair_body, 0)

        @pl.when(nch64 > 0)
        def _():
            pltpu.make_async_copy(rows_a, out64(0), semo_a).wait()

        @pl.when((nch64 > 0) & (nch64 == (nch64 // 2) * 2))
        def _():
            pltpu.make_async_copy(rows_b, out64(0), semo_b).wait()

        j0 = nch64 * 64

        def body16(j, carry):
            t_vec = (base + j0 + j * 16) + lane16
            b_vec = jnp.zeros((16,), jnp.int32)
            for jj in range(B):
                b_vec = b_vec + (t_vec >= cum_hi[jj]).astype(jnp.int32)
            b_vec = jnp.minimum(b_vec, B - 1)
            st = plsc.load_gather(aux_v, [b_vec])
            cl = plsc.load_gather(aux_v, [b_vec + 16])
            pos = st + (t_vec - cl) + b_vec * S
            pos = jnp.where(t_vec < t_end, pos, 0)
            ids_t[...] = plsc.load_gather(ctx_v, [pos])
            rows_t = rows_a.at[pl.ds(0, 16)]
            pltpu.async_copy(emb_hbm.at[ids_t], rows_t, semg_a).wait()
            pltpu.sync_copy(
                rows_t,
                tok_out.at[pl.ds(pl.multiple_of(base + j0 + j * 16, 16), 16)])
            return carry

        lax.fori_loop(0, rem16, body16, 0)

        pltpu.make_async_copy(sw_hbm.at[sidx_v], srows_v, sem_sw).wait()
        pltpu.sync_copy(srows_v, wg_out.at[pl.ds(sbase, 16)])
        pltpu.make_async_copy(sb_hbm.at[sidx_hi_v], sbrows_v, sem_sb).wait()
        lane = lax.iota(jnp.int32, 16)
        sb_v[...] = plsc.load_gather(sbrows_v, [lane, jnp.bitwise_and(sids, 127)])
        pltpu.sync_copy(sb_v, bg_out.at[pl.ds(sbase, 16)])

    return sc_gather


def _sc_gather(*args):
    return _make_sc_gather()(*args)


def _num_blocks(sp_ref):
    t_total = jnp.int32(0)
    for b in range(B):
        s_b = sp_ref[2 * b]
        e_b = jnp.maximum(sp_ref[2 * b + 1], s_b + 1)
        t_total = t_total + (e_b - s_b)
    return (t_total + (BLK - 1)) // BLK


def _tc_body(sp_ref, tok_ref, w_ref, b_ref, lohi_ref, iw_ref,
             wg_ref, bgr_ref, tgt_ref, loss_ref, corr_ref, acc_ref):
    i = pl.program_id(0)
    nb = _num_blocks(sp_ref)

    @pl.when(i == 0)
    def _():
        acc_ref[...] = jnp.zeros_like(acc_ref)

    @pl.when(i < nb)
    def _():
        h = jnp.tanh(
            jnp.dot(tok_ref[...], w_ref[...], preferred_element_type=jnp.float32)
            + b_ref[...])
        gcol = i * BLK + lax.broadcasted_iota(jnp.int32, (B, BLK), 1)
        lo = lohi_ref[:, 0:1]
        hi = lohi_ref[:, 1:2]
        m = ((gcol >= lo) & (gcol < hi)).astype(jnp.float32)
        m = m * iw_ref[...]
        acc_ref[...] += jnp.dot(m, h, preferred_element_type=jnp.float32)

    @pl.when(i == NBLK - 1)
    def _():
        reps = acc_ref[...]
        rows = []
        for b in range(B):
            wgb = wg_ref[pl.ds(b * NCAND, NCAND), :]
            rb = reps[b:b + 1, :]
            rows.append(lax.dot_general(
                rb, wgb, (((1,), (1,)), ((), ())),
                preferred_element_type=jnp.float32))
        logits = jnp.concatenate(rows, axis=0) + bgr_ref[...]

        mx = jnp.max(logits, axis=1, keepdims=True)
        ex = jnp.exp(logits - mx)
        z = jnp.sum(ex, axis=1, keepdims=True)
        logz = jnp.log(z) + mx
        ci = lax.broadcasted_iota(jnp.int32, (B, NCAND), 1)
        tgt = tgt_ref[...]
        tl = jnp.sum(jnp.where(ci == tgt, logits, 0.0), axis=1, keepdims=True)
        loss_ref[...] = jnp.sum((logz - tl) * (1.0 / B), axis=0, keepdims=True)
        amax = jnp.min(jnp.where(logits == mx, ci, NCAND), axis=1, keepdims=True)
        corr_ref[...] = (amax == tgt).astype(jnp.int32)


def _tc_forward(spans_flat, tok, w_enc, b_enc2, lohi, iw, wg, bgr, tgt2):
    grid_spec = pltpu.PrefetchScalarGridSpec(
        num_scalar_prefetch=1,
        grid=(NBLK,),
        in_specs=[
            pl.BlockSpec((BLK, D),
                         lambda i, sp: (jnp.minimum(i, _num_blocks(sp) - 1), 0)),
            pl.BlockSpec((D, D), lambda i, sp: (0, 0)),
            pl.BlockSpec((1, D), lambda i, sp: (0, 0)),
            pl.BlockSpec((B, 2), lambda i, sp: (0, 0)),
            pl.BlockSpec((B, 1), lambda i, sp: (0, 0)),
            pl.BlockSpec((B * NCAND, D), lambda i, sp: (0, 0)),
            pl.BlockSpec((B, NCAND), lambda i, sp: (0, 0)),
            pl.BlockSpec((B, 1), lambda i, sp: (0, 0)),
        ],
        out_specs=[
            pl.BlockSpec((1, 1), lambda i, sp: (0, 0)),
            pl.BlockSpec((B, 1), lambda i, sp: (0, 0)),
        ],
        scratch_shapes=[pltpu.VMEM((B, D), jnp.float32)],
    )
    return pl.pallas_call(
        _tc_body,
        grid_spec=grid_spec,
        out_shape=[
            jax.ShapeDtypeStruct((1, 1), jnp.float32),
            jax.ShapeDtypeStruct((B, 1), jnp.int32),
        ],
    )(spans_flat, tok, w_enc, b_enc2, lohi, iw, wg, bgr, tgt2)


def kernel(context_ids, context_spans, sense_ids, target_ids, emb_table,
           W_enc, b_enc, sense_W, sense_b):
    context_ids = context_ids.astype(jnp.int32)
    context_spans = context_spans.astype(jnp.int32)
    sense_ids = sense_ids.astype(jnp.int32)
    target_ids = target_ids.astype(jnp.int32)

    spans_flat = context_spans.reshape(-1)
    ctx_flat = context_ids.reshape(-1)
    sids_flat = sense_ids.reshape(-1)

    n_senses = sense_b.shape[0]
    pad_b = (-n_senses) % 128
    sb_rows = jnp.pad(sense_b, (0, pad_b)).reshape(-1, 128)
    tok, wg, bg, lohi, iw = _sc_gather(ctx_flat, spans_flat, sids_flat,
                                       emb_table, sense_W, sb_rows)

    loss2, corr2 = _tc_forward(spans_flat, tok, W_enc, b_enc.reshape(1, D),
                               lohi, iw, wg, bg.reshape(B, NCAND),
                               target_ids.reshape(B, 1))
    return loss2[0, 0], corr2[:, 0].astype(jnp.bool_)
